# baseline (speedup 1.0000x reference)
import functools

import jax
import jax.numpy as jnp
from jax.experimental import pallas as pl
from jax.experimental.pallas import tpu as pltpu


def _gru_body(mem_ref, agg_ref, wih_ref, whh_ref, bih_ref, bhh_ref, out_ref):
    mem = mem_ref[...]
    agg = agg_ref[...]
    gi = jnp.dot(agg, wih_ref[...], preferred_element_type=jnp.float32)
    gi = gi + bih_ref[...][None, :]
    gh = jnp.dot(mem, whh_ref[...], preferred_element_type=jnp.float32)
    gh = gh + bhh_ref[...][None, :]
    D = mem.shape[1]
    i_r, i_z, i_n = gi[:, :D], gi[:, D:2 * D], gi[:, 2 * D:]
    h_r, h_z, h_n = gh[:, :D], gh[:, D:2 * D], gh[:, 2 * D:]
    r = jax.nn.sigmoid(i_r + h_r)
    z = jax.nn.sigmoid(i_z + h_z)
    n = jnp.tanh(i_n + r * h_n)
    out_ref[...] = (1.0 - z) * n + z * mem


def _tc_gru(h_e, agg_e, W_ih, W_hh, b_ih, b_hh):
    B, D = h_e.shape
    BLK = 2048
    return pl.pallas_call(
        _gru_body,
        grid=(B // BLK,),
        in_specs=[
            pl.BlockSpec((BLK, D), lambda i: (i, 0)),
            pl.BlockSpec((BLK, D), lambda i: (i, 0)),
            pl.BlockSpec((D, 3 * D), lambda i: (0, 0)),
            pl.BlockSpec((D, 3 * D), lambda i: (0, 0)),
            pl.BlockSpec((3 * D,), lambda i: (0,)),
            pl.BlockSpec((3 * D,), lambda i: (0,)),
        ],
        out_specs=pl.BlockSpec((BLK, D), lambda i: (i, 0)),
        out_shape=jax.ShapeDtypeStruct((B, D), jnp.float32),
    )(h_e, agg_e, W_ih, W_hh, b_ih, b_hh)


def kernel(mem, idx, val, t, W_ih, W_hh, b_ih, b_hh):
    Mn, Dn = mem.shape
    B = idx.shape[0]
    t_max = jax.ops.segment_max(t, idx, num_segments=Mn)
    w = jnp.exp(t - t_max[idx])
    w_sum = jax.ops.segment_sum(w, idx, num_segments=Mn)
    rep = jax.ops.segment_max(jnp.arange(B, dtype=jnp.int32), idx,
                              num_segments=Mn)
    rep_e = rep[idx]
    acc = jnp.zeros((B, Dn), jnp.float32).at[rep_e].add(w[:, None] * val)
    agg_e = acc[rep_e] / jnp.maximum(w_sum[idx], 1e-6)[:, None]
    h_e = mem[idx]
    new_h = _tc_gru(h_e, agg_e, W_ih, W_hh, b_ih, b_hh)
    return mem.at[idx].set(new_h)


# speedup vs baseline: 1.1788x; 1.1788x over previous
import functools

import jax
import jax.numpy as jnp
from jax import lax
from jax.experimental import pallas as pl
from jax.experimental.pallas import tpu as pltpu
from jax.experimental.pallas import tpu_sc as plsc

_SC_PARAMS = pltpu.CompilerParams(use_tc_tiling_on_sc=False)
_NW = 32
_CH = 128
_HALF = 1024


def _sc_gather_rows(table, idx):
    B, = idx.shape
    M, D = table.shape
    BPW = B // _NW
    mesh = plsc.VectorSubcoreMesh(core_axis_name="c", subcore_axis_name="s")

    @functools.partial(
        pl.kernel, mesh=mesh,
        out_type=jax.ShapeDtypeStruct((B, D), jnp.float32),
        scratch_types=[
            pltpu.VMEM((BPW,), jnp.int32),
            pltpu.VMEM((_HALF, D), jnp.float32),
            pltpu.SemaphoreType.DMA,
        ],
        compiler_params=_SC_PARAMS,
    )
    def k(table_hbm, idx_hbm, out_hbm, idx_v, rows_v, sem):
        wid = lax.axis_index("s") * 2 + lax.axis_index("c")
        base = wid * BPW
        pltpu.sync_copy(idx_hbm.at[pl.ds(base, BPW)], idx_v)
        for h in range(BPW // _HALF):
            for j in range(_HALF // _CH):
                pltpu.async_copy(
                    table_hbm.at[idx_v.at[pl.ds(h * _HALF + j * _CH, _CH)]],
                    rows_v.at[pl.ds(j * _CH, _CH)], sem)
            for j in range(_HALF // _CH):
                pltpu.make_async_copy(
                    table_hbm.at[idx_v.at[pl.ds(h * _HALF + j * _CH, _CH)]],
                    rows_v.at[pl.ds(j * _CH, _CH)], sem).wait()
            pltpu.sync_copy(rows_v, out_hbm.at[pl.ds(base + h * _HALF, _HALF)])

    return k(table, idx)


def _sc_scatter_rows(idx, rows, out_ref):
    B, D = rows.shape
    BPW = B // _NW
    NJ = BPW // _CH
    mesh = plsc.VectorSubcoreMesh(core_axis_name="c", subcore_axis_name="s")

    @functools.partial(
        pl.kernel, mesh=mesh,
        out_type=(),
        scratch_types=[
            pltpu.VMEM((NJ, _CH), jnp.int32),
            pltpu.VMEM((_HALF, D), jnp.float32),
            pltpu.SemaphoreType.DMA,
        ],
        compiler_params=_SC_PARAMS,
    )
    def k(idx_hbm, rows_hbm, out_hbm, idx_v, rows_v, sem):
        wid = lax.axis_index("s") * 2 + lax.axis_index("c")
        base = wid * BPW
        for j in range(NJ):
            pltpu.sync_copy(idx_hbm.at[pl.ds(base + j * _CH, _CH)], idx_v.at[j])
        per_half = _HALF // _CH
        for h in range(BPW // _HALF):
            pltpu.sync_copy(rows_hbm.at[pl.ds(base + h * _HALF, _HALF)], rows_v)
            for j in range(per_half):
                pltpu.async_copy(
                    rows_v.at[pl.ds(j * _CH, _CH)],
                    out_hbm.at[idx_v.at[h * per_half + j]], sem)
            for j in range(per_half):
                pltpu.make_async_copy(
                    rows_v.at[pl.ds(j * _CH, _CH)],
                    out_hbm.at[idx_v.at[h * per_half + j]], sem).wait()

    k(idx, rows, out_ref)


def _gru_body(mem_ref, agg_ref, wih_ref, whh_ref, bih_ref, bhh_ref, out_ref):
    mem = mem_ref[...]
    agg = agg_ref[...]
    gi = jnp.dot(agg, wih_ref[...], preferred_element_type=jnp.float32)
    gi = gi + bih_ref[...][None, :]
    gh = jnp.dot(mem, whh_ref[...], preferred_element_type=jnp.float32)
    gh = gh + bhh_ref[...][None, :]
    D = mem.shape[1]
    i_r, i_z, i_n = gi[:, :D], gi[:, D:2 * D], gi[:, 2 * D:]
    h_r, h_z, h_n = gh[:, :D], gh[:, D:2 * D], gh[:, 2 * D:]
    r = jax.nn.sigmoid(i_r + h_r)
    z = jax.nn.sigmoid(i_z + h_z)
    n = jnp.tanh(i_n + r * h_n)
    out_ref[...] = (1.0 - z) * n + z * mem


def _tc_gru(h_e, agg_e, W_ih, W_hh, b_ih, b_hh):
    B, D = h_e.shape
    BLK = 2048
    return pl.pallas_call(
        _gru_body,
        grid=(B // BLK,),
        in_specs=[
            pl.BlockSpec((BLK, D), lambda i: (i, 0)),
            pl.BlockSpec((BLK, D), lambda i: (i, 0)),
            pl.BlockSpec((D, 3 * D), lambda i: (0, 0)),
            pl.BlockSpec((D, 3 * D), lambda i: (0, 0)),
            pl.BlockSpec((3 * D,), lambda i: (0,)),
            pl.BlockSpec((3 * D,), lambda i: (0,)),
        ],
        out_specs=pl.BlockSpec((BLK, D), lambda i: (i, 0)),
        out_shape=jax.ShapeDtypeStruct((B, D), jnp.float32),
    )(h_e, agg_e, W_ih, W_hh, b_ih, b_hh)


def kernel(mem, idx, val, t, W_ih, W_hh, b_ih, b_hh):
    Mn, Dn = mem.shape
    B = idx.shape[0]
    t_max = jax.ops.segment_max(t, idx, num_segments=Mn)
    w = jnp.exp(t - t_max[idx])
    w_sum = jax.ops.segment_sum(w, idx, num_segments=Mn)
    rep = jax.ops.segment_max(jnp.arange(B, dtype=jnp.int32), idx,
                              num_segments=Mn)
    rep_e = rep[idx]
    acc = jnp.zeros((B, Dn), jnp.float32).at[rep_e].add(w[:, None] * val)
    agg_e = acc[rep_e] / jnp.maximum(w_sum[idx], 1e-6)[:, None]
    h_e = _sc_gather_rows(mem, idx)
    new_h = _tc_gru(h_e, agg_e, W_ih, W_hh, b_ih, b_hh)
    out_ref = jax.new_ref(mem)
    _sc_scatter_rows(idx, new_h, out_ref)
    return out_ref[...]


# speedup vs baseline: 1.9060x; 1.6169x over previous
import functools

import jax
import jax.numpy as jnp
from jax import lax
from jax.experimental import pallas as pl
from jax.experimental.pallas import tpu as pltpu
from jax.experimental.pallas import tpu_sc as plsc

_SC_PARAMS = pltpu.CompilerParams(use_tc_tiling_on_sc=False,
                                  needs_layout_passes=False)
_NW = 32
_NT = 16
_CH = 128
_HALF = 1024
_NBIN = 16
_BINW = 64.0


def _mesh():
    return plsc.VectorSubcoreMesh(core_axis_name="c", subcore_axis_name="s")


def _wid():
    return lax.axis_index("s") * 2 + lax.axis_index("c")


def _sc_rep_scatter(idx, Mn):
    B, = idx.shape
    BPW = B // _NW
    NJ = BPW // _CH

    @functools.partial(
        pl.kernel, mesh=_mesh(),
        out_type=jax.ShapeDtypeStruct((Mn,), jnp.int32),
        scratch_types=[
            pltpu.VMEM((NJ, _CH), jnp.int32),
            pltpu.VMEM((BPW,), jnp.int32),
            pltpu.SemaphoreType.DMA,
        ],
        compiler_params=_SC_PARAMS,
    )
    def k(idx_hbm, rep_hbm, idx2_v, eid_v, sem):
        base = _wid() * BPW

        def fill(i, _):
            eid_v[pl.ds(i * 16, 16)] = lax.iota(jnp.int32, 16) + (base + i * 16)
            return 0
        lax.fori_loop(0, BPW // 16, fill, 0)
        for j in range(NJ):
            pltpu.sync_copy(idx_hbm.at[pl.ds(base + j * _CH, _CH)],
                            idx2_v.at[j])
        for j in range(NJ):
            pltpu.async_copy(eid_v.at[pl.ds(j * _CH, _CH)],
                             rep_hbm.at[idx2_v.at[j]], sem)
        for j in range(NJ):
            pltpu.make_async_copy(eid_v.at[pl.ds(j * _CH, _CH)],
                                  rep_hbm.at[idx2_v.at[j]], sem).wait()

    return k(idx)


def _sc_bins(idx, t, rep_table):
    B, = idx.shape
    BPW = B // _NW
    NJ = BPW // _CH
    TBL = B * _NBIN
    SH = TBL // _NT
    ZB = 4096

    @functools.partial(
        pl.kernel, mesh=_mesh(),
        out_type=[jax.ShapeDtypeStruct((B,), jnp.int32),
                  jax.ShapeDtypeStruct((2, TBL), jnp.float32)],
        scratch_types=[
            pltpu.VMEM((BPW,), jnp.int32),
            pltpu.VMEM((BPW,), jnp.float32),
            pltpu.VMEM((BPW,), jnp.int32),
            pltpu.VMEM((NJ, _CH), jnp.int32),
            pltpu.VMEM((BPW,), jnp.float32),
            pltpu.VMEM((ZB,), jnp.float32),
            pltpu.VMEM_SHARED((TBL,), jnp.float32),
            pltpu.SemaphoreType.DMA,
        ],
        compiler_params=_SC_PARAMS,
    )
    def k(idx_hbm, t_hbm, rept_hbm, repe_hbm, sbin_hbm,
          idx_v, t_v, rep_v, fi2_v, cb_v, zb_v, shared, sem):
        c = lax.axis_index("c")
        s = lax.axis_index("s")
        base = (c * _NT + s) * BPW

        def zfill(i, _):
            zb_v[pl.ds(i * 16, 16)] = jnp.zeros((16,), jnp.float32)
            return 0
        lax.fori_loop(0, ZB // 16, zfill, 0)

        def zcp(i, _):
            pltpu.sync_copy(zb_v, shared.at[pl.ds(s * SH + i * ZB, ZB)])
            return 0
        lax.fori_loop(0, SH // ZB, zcp, 0)

        pltpu.sync_copy(idx_hbm.at[pl.ds(base, BPW)], idx_v)
        pltpu.sync_copy(t_hbm.at[pl.ds(base, BPW)], t_v)
        for j in range(NJ):
            pltpu.async_copy(rept_hbm.at[idx_v.at[pl.ds(j * _CH, _CH)]],
                             rep_v.at[pl.ds(j * _CH, _CH)], sem)
        for j in range(NJ):
            pltpu.make_async_copy(rept_hbm.at[idx_v.at[pl.ds(j * _CH, _CH)]],
                                  rep_v.at[pl.ds(j * _CH, _CH)], sem).wait()
        pltpu.sync_copy(rep_v, repe_hbm.at[pl.ds(base, BPW)])

        def compute(j, _):
            def inner(v, _):
                o = j * _CH + v * 16
                tv = t_v[pl.ds(o, 16)]
                rv = rep_v[pl.ds(o, 16)]
                q = jnp.minimum((tv * (1.0 / _BINW)).astype(jnp.int32),
                                _NBIN - 1)
                cb = jnp.exp(tv - q.astype(jnp.float32) * _BINW)
                fi2_v[j, pl.ds(v * 16, 16)] = rv * _NBIN + q
                cb_v[pl.ds(o, 16)] = cb
                return 0
            lax.fori_loop(0, _CH // 16, inner, 0)
            return 0
        lax.fori_loop(0, NJ, compute, 0)

        plsc.subcore_barrier()
        for j in range(NJ):
            pltpu.sync_copy(cb_v.at[pl.ds(j * _CH, _CH)],
                            shared.at[fi2_v.at[j]], add=True)
        plsc.subcore_barrier()

        def dump(i, _):
            pltpu.sync_copy(shared.at[pl.ds(s * SH + i * ZB, ZB)], zb_v)
            pltpu.sync_copy(zb_v, sbin_hbm.at[c, pl.ds(s * SH + i * ZB, ZB)])
            return 0
        lax.fori_loop(0, SH // ZB, dump, 0)

    return k(idx, t, rep_table)


def _sc_bin_reduce(sbin, B):
    SPW = B // _NW
    CK = 256

    @functools.partial(
        pl.kernel, mesh=_mesh(),
        out_type=[jax.ShapeDtypeStruct((B,), jnp.float32),
                  jax.ShapeDtypeStruct((B,), jnp.float32)],
        scratch_types=[
            pltpu.VMEM((CK * _NBIN,), jnp.float32),
            pltpu.VMEM((CK * _NBIN,), jnp.float32),
            pltpu.VMEM((SPW,), jnp.float32),
            pltpu.VMEM((SPW,), jnp.float32),
            pltpu.SemaphoreType.DMA,
        ],
        compiler_params=_SC_PARAMS,
    )
    def k(sbin_hbm, inv_hbm, c_hbm, b0_v, b1_v, inv_v, c_v, sem):
        base = _wid() * SPW
        lane16 = lax.iota(jnp.int32, 16) * _NBIN

        def chunk(ch, _):
            off = (base + ch * CK) * _NBIN
            pltpu.sync_copy(sbin_hbm.at[0, pl.ds(off, CK * _NBIN)], b0_v)
            pltpu.sync_copy(sbin_hbm.at[1, pl.ds(off, CK * _NBIN)], b1_v)

            def group(g, _):
                goff = lane16 + g * (16 * _NBIN)
                svals = []
                for kk in range(_NBIN):
                    svals.append(plsc.load_gather(b0_v, [goff + kk]) +
                                 plsc.load_gather(b1_v, [goff + kk]))
                kmax = jnp.full((16,), -1, jnp.int32)
                for kk in range(_NBIN):
                    kmax = jnp.where(svals[kk] > 0.0, kk, kmax)
                kmf = jnp.maximum(kmax, 0).astype(jnp.float32)
                wsum = jnp.zeros((16,), jnp.float32)
                for kk in range(_NBIN):
                    e = jnp.exp(jnp.minimum((kk - kmf) * _BINW, 0.0))
                    wsum = wsum + svals[kk] * e
                inv_v[pl.ds(ch * CK + g * 16, 16)] = (
                    1.0 / jnp.maximum(wsum, 1e-6))
                c_v[pl.ds(ch * CK + g * 16, 16)] = kmf * _BINW
                return 0
            lax.fori_loop(0, CK // 16, group, 0)
            return 0
        lax.fori_loop(0, SPW // CK, chunk, 0)
        pltpu.sync_copy(inv_v, inv_hbm.at[pl.ds(base, SPW)])
        pltpu.sync_copy(c_v, c_hbm.at[pl.ds(base, SPW)])

    return k(sbin)


def _sc_we(t, rep_e, c_tab):
    B, = t.shape
    BPW = B // _NW
    NJ = BPW // _CH

    @functools.partial(
        pl.kernel, mesh=_mesh(),
        out_type=jax.ShapeDtypeStruct((B,), jnp.float32),
        scratch_types=[
            pltpu.VMEM((BPW,), jnp.float32),
            pltpu.VMEM((BPW,), jnp.int32),
            pltpu.VMEM((BPW,), jnp.float32),
            pltpu.SemaphoreType.DMA,
        ],
        compiler_params=_SC_PARAMS,
    )
    def k(t_hbm, repe_hbm, c_hbm, we_hbm, t_v, rep_v, c_v, sem):
        base = _wid() * BPW
        pltpu.sync_copy(t_hbm.at[pl.ds(base, BPW)], t_v)
        pltpu.sync_copy(repe_hbm.at[pl.ds(base, BPW)], rep_v)
        for j in range(NJ):
            pltpu.async_copy(c_hbm.at[rep_v.at[pl.ds(j * _CH, _CH)]],
                             c_v.at[pl.ds(j * _CH, _CH)], sem)
        for j in range(NJ):
            pltpu.make_async_copy(c_hbm.at[rep_v.at[pl.ds(j * _CH, _CH)]],
                                  c_v.at[pl.ds(j * _CH, _CH)], sem).wait()

        def body(i, _):
            o = i * 16
            t_v[pl.ds(o, 16)] = jnp.exp(t_v[pl.ds(o, 16)] - c_v[pl.ds(o, 16)])
            return 0
        lax.fori_loop(0, BPW // 16, body, 0)
        pltpu.sync_copy(t_v, we_hbm.at[pl.ds(base, BPW)])

    return k(t, rep_e, c_tab)


def _sc_macc(val, rep_e, w_e):
    B, D = val.shape
    EPS = B // _NT
    NJ = EPS // _CH
    CG = 16
    NR = D // (2 * CG)
    CHUNK = 1024
    ZB = 4096

    @functools.partial(
        pl.kernel, mesh=_mesh(),
        out_type=jax.ShapeDtypeStruct((B, D), jnp.float32),
        scratch_types=[
            pltpu.VMEM((NJ, _CH), jnp.int32),
            pltpu.VMEM((EPS,), jnp.float32),
            pltpu.VMEM((CHUNK, CG), jnp.float32),
            pltpu.VMEM((ZB // CG, CG), jnp.float32),
            pltpu.VMEM((EPS // _NT, CG), jnp.float32),
            pltpu.VMEM_SHARED((B, CG), jnp.float32),
            pltpu.SemaphoreType.DMA,
        ],
        compiler_params=_SC_PARAMS,
    )
    def k(val_hbm, repe_hbm, we_hbm, macc_hbm,
          fi2_v, w_v, vb_v, zb_v, db_v, shared, sem):
        c = lax.axis_index("c")
        s = lax.axis_index("s")
        ebase = s * EPS
        ZROWS = ZB // CG
        TROWS = B // _NT

        def fill_fi(j, _):
            pltpu.sync_copy(repe_hbm.at[pl.ds(ebase + j * _CH, _CH)],
                            fi2_v.at[j])
            return 0
        lax.fori_loop(0, NJ, fill_fi, 0)
        pltpu.sync_copy(we_hbm.at[pl.ds(ebase, EPS)], w_v)

        def zfill(i, _):
            zb_v[i, :] = jnp.zeros((CG,), jnp.float32)
            return 0
        lax.fori_loop(0, ZROWS, zfill, 0)

        for r in range(NR):
            g = c * (NR * CG) + r * CG

            def zcp(i, _):
                pltpu.sync_copy(zb_v,
                                shared.at[pl.ds(s * TROWS + i * ZROWS,
                                                ZROWS)])
                return 0
            lax.fori_loop(0, TROWS // ZROWS, zcp, 0)
            plsc.subcore_barrier()

            def chunk(ch, _):
                pltpu.sync_copy(
                    val_hbm.at[pl.ds(ebase + ch * CHUNK, CHUNK),
                               pl.ds(g, CG)], vb_v)

                def scale(i2, _):
                    wv = w_v[pl.ds(ch * CHUNK + i2 * 16, 16)]
                    for l in range(16):
                        vb_v[i2 * 16 + l, :] = vb_v[i2 * 16 + l, :] * wv[l]
                    return 0
                lax.fori_loop(0, CHUNK // 16, scale, 0)
                for j in range(CHUNK // _CH):
                    pltpu.sync_copy(
                        vb_v.at[pl.ds(j * _CH, _CH)],
                        shared.at[fi2_v.at[ch * (CHUNK // _CH) + j]],
                        add=True)
                return 0
            lax.fori_loop(0, EPS // CHUNK, chunk, 0)
            plsc.subcore_barrier()

            DROWS = EPS // _NT

            def dump(i, _):
                off = s * TROWS + i * DROWS
                pltpu.sync_copy(shared.at[pl.ds(off, DROWS)], db_v)
                pltpu.sync_copy(db_v,
                                macc_hbm.at[pl.ds(off, DROWS), pl.ds(g, CG)])
                return 0
            lax.fori_loop(0, TROWS // DROWS, dump, 0)
            if r + 1 < NR:
                plsc.subcore_barrier()

    return k(val, rep_e, w_e)


def _sc_gather3(mem, idx, macc, rep_e, inv_tab):
    Mn, D = mem.shape
    B, = idx.shape
    BPW = B // _NW
    NJ = BPW // _CH

    @functools.partial(
        pl.kernel, mesh=_mesh(),
        out_type=[jax.ShapeDtypeStruct((B, D), jnp.float32),
                  jax.ShapeDtypeStruct((B, D), jnp.float32)],
        scratch_types=[
            pltpu.VMEM((BPW,), jnp.int32),
            pltpu.VMEM((BPW,), jnp.int32),
            pltpu.VMEM((BPW,), jnp.float32),
            pltpu.VMEM((_HALF, D), jnp.float32),
            pltpu.SemaphoreType.DMA,
        ],
        compiler_params=_SC_PARAMS,
    )
    def k(mem_hbm, idx_hbm, macc_hbm, repe_hbm, inv_hbm,
          he_hbm, agg_hbm, idx_v, rep_v, inv_v, rows_v, sem):
        base = _wid() * BPW
        pltpu.sync_copy(idx_hbm.at[pl.ds(base, BPW)], idx_v)
        pltpu.sync_copy(repe_hbm.at[pl.ds(base, BPW)], rep_v)
        for j in range(NJ):
            pltpu.async_copy(inv_hbm.at[rep_v.at[pl.ds(j * _CH, _CH)]],
                             inv_v.at[pl.ds(j * _CH, _CH)], sem)
        for j in range(NJ):
            pltpu.make_async_copy(inv_hbm.at[rep_v.at[pl.ds(j * _CH, _CH)]],
                                  inv_v.at[pl.ds(j * _CH, _CH)], sem).wait()

        for h in range(BPW // _HALF):
            for j in range(_HALF // _CH):
                o = h * _HALF + j * _CH
                pltpu.async_copy(mem_hbm.at[idx_v.at[pl.ds(o, _CH)]],
                                 rows_v.at[pl.ds(j * _CH, _CH)], sem)
            for j in range(_HALF // _CH):
                o = h * _HALF + j * _CH
                pltpu.make_async_copy(mem_hbm.at[idx_v.at[pl.ds(o, _CH)]],
                                      rows_v.at[pl.ds(j * _CH, _CH)],
                                      sem).wait()
            pltpu.sync_copy(rows_v, he_hbm.at[pl.ds(base + h * _HALF, _HALF)])

        for h in range(BPW // _HALF):
            for j in range(_HALF // _CH):
                o = h * _HALF + j * _CH
                pltpu.async_copy(macc_hbm.at[rep_v.at[pl.ds(o, _CH)]],
                                 rows_v.at[pl.ds(j * _CH, _CH)], sem)
            for j in range(_HALF // _CH):
                o = h * _HALF + j * _CH
                pltpu.make_async_copy(macc_hbm.at[rep_v.at[pl.ds(o, _CH)]],
                                      rows_v.at[pl.ds(j * _CH, _CH)],
                                      sem).wait()

            def scale(i2, _):
                wv = inv_v[pl.ds(h * _HALF + i2 * 16, 16)]
                for l in range(16):
                    for u in range(D // 16):
                        rows_v[i2 * 16 + l, pl.ds(u * 16, 16)] = (
                            rows_v[i2 * 16 + l, pl.ds(u * 16, 16)] * wv[l])
                return 0
            lax.fori_loop(0, _HALF // 16, scale, 0)
            pltpu.sync_copy(rows_v, agg_hbm.at[pl.ds(base + h * _HALF, _HALF)])

    return k(mem, idx, macc, rep_e, inv_tab)


def _sc_scatter_rows(idx, rows, out_ref):
    B, D = rows.shape
    BPW = B // _NW
    NJ = BPW // _CH

    @functools.partial(
        pl.kernel, mesh=_mesh(),
        out_type=(),
        scratch_types=[
            pltpu.VMEM((NJ, _CH), jnp.int32),
            pltpu.VMEM((_HALF, D), jnp.float32),
            pltpu.SemaphoreType.DMA,
        ],
        compiler_params=_SC_PARAMS,
    )
    def k(idx_hbm, rows_hbm, out_hbm, idx_v, rows_v, sem):
        base = _wid() * BPW
        for j in range(NJ):
            pltpu.sync_copy(idx_hbm.at[pl.ds(base + j * _CH, _CH)], idx_v.at[j])
        per_half = _HALF // _CH
        for h in range(BPW // _HALF):
            pltpu.sync_copy(rows_hbm.at[pl.ds(base + h * _HALF, _HALF)], rows_v)
            for j in range(per_half):
                pltpu.async_copy(rows_v.at[pl.ds(j * _CH, _CH)],
                                 out_hbm.at[idx_v.at[h * per_half + j]], sem)
            for j in range(per_half):
                pltpu.make_async_copy(rows_v.at[pl.ds(j * _CH, _CH)],
                                      out_hbm.at[idx_v.at[h * per_half + j]],
                                      sem).wait()

    k(idx, rows, out_ref)


def _gru_body(mem_ref, agg_ref, wih_ref, whh_ref, bih_ref, bhh_ref, out_ref):
    mem = mem_ref[...]
    agg = agg_ref[...]
    gi = jnp.dot(agg, wih_ref[...], preferred_element_type=jnp.float32)
    gi = gi + bih_ref[...][None, :]
    gh = jnp.dot(mem, whh_ref[...], preferred_element_type=jnp.float32)
    gh = gh + bhh_ref[...][None, :]
    D = mem.shape[1]
    i_r, i_z, i_n = gi[:, :D], gi[:, D:2 * D], gi[:, 2 * D:]
    h_r, h_z, h_n = gh[:, :D], gh[:, D:2 * D], gh[:, 2 * D:]
    r = jax.nn.sigmoid(i_r + h_r)
    z = jax.nn.sigmoid(i_z + h_z)
    n = jnp.tanh(i_n + r * h_n)
    out_ref[...] = (1.0 - z) * n + z * mem


def _tc_gru(h_e, agg_e, W_ih, W_hh, b_ih, b_hh):
    B, D = h_e.shape
    BLK = 2048
    return pl.pallas_call(
        _gru_body,
        grid=(B // BLK,),
        in_specs=[
            pl.BlockSpec((BLK, D), lambda i: (i, 0)),
            pl.BlockSpec((BLK, D), lambda i: (i, 0)),
            pl.BlockSpec((D, 3 * D), lambda i: (0, 0)),
            pl.BlockSpec((D, 3 * D), lambda i: (0, 0)),
            pl.BlockSpec((3 * D,), lambda i: (0,)),
            pl.BlockSpec((3 * D,), lambda i: (0,)),
        ],
        out_specs=pl.BlockSpec((BLK, D), lambda i: (i, 0)),
        out_shape=jax.ShapeDtypeStruct((B, D), jnp.float32),
    )(h_e, agg_e, W_ih, W_hh, b_ih, b_hh)


def kernel(mem, idx, val, t, W_ih, W_hh, b_ih, b_hh):
    Mn, Dn = mem.shape
    B = idx.shape[0]
    idx = idx.astype(jnp.int32)
    rep_table = _sc_rep_scatter(idx, Mn)
    rep_e, sbin = _sc_bins(idx, t, rep_table)
    inv_tab, c_tab = _sc_bin_reduce(sbin, B)
    w_e = _sc_we(t, rep_e, c_tab)
    macc = _sc_macc(val, rep_e, w_e)
    h_e, agg_e = _sc_gather3(mem, idx, macc, rep_e, inv_tab)
    new_h = _tc_gru(h_e, agg_e, W_ih, W_hh, b_ih, b_hh)
    out_ref = jax.new_ref(mem)
    _sc_scatter_rows(idx, new_h, out_ref)
    return out_ref[...]
